# Initial kernel scaffold; baseline (speedup 1.0000x reference)
import jax
import jax.numpy as jnp
from jax.experimental import pallas as pl


def kernel(x, edge_index, batch, edge_attr, emb_table, bn_gamma0, bn_beta0, W0, b0, bn_gamma1, bn_beta1, W1, b1):
    raise NotImplementedError("write your pallas kernel here")


# speedup vs baseline: 16.9066x; 16.9066x over previous
import functools

import jax
import jax.numpy as jnp
from jax import lax
from jax.experimental import pallas as pl
from jax.experimental.pallas import tpu as pltpu
from jax.experimental.pallas import tpu_sc as plsc

N = 10000
E = 320000
V = 100000
D = 128
G = 256

NPAD = 10240
NW = 32
RPW = NPAD // NW
RPS = NPAD // 16
EPW = E // NW
L = 16
NGA = EPW // L
KEB = 80
NGB = EPW // KEB


def _mesh():
    return plsc.VectorSubcoreMesh(core_axis_name="c", subcore_axis_name="s")


_SC_PARAMS = pltpu.CompilerParams(needs_layout_passes=False)


@functools.partial(
    pl.kernel,
    out_type=(
        jax.ShapeDtypeStruct((NPAD, D), jnp.float32),
        jax.ShapeDtypeStruct((NW * NPAD,), jnp.float32),
    ),
    mesh=_mesh(),
    scratch_types=[
        pltpu.VMEM((RPW,), jnp.int32),
        pltpu.VMEM((RPW, D), jnp.float32),
        pltpu.VMEM((EPW,), jnp.int32),
        pltpu.VMEM((EPW,), jnp.float32),
        pltpu.VMEM((NPAD,), jnp.float32),
        pltpu.SemaphoreType.DMA,
    ],
    compiler_params=_SC_PARAMS,
)
def _emb_deg(xidx_hbm, dst_hbm, attr_hbm, emb_hbm, h0_hbm, degp_hbm,
             idx_v, rows_v, dst_v, attr_v, deg_v, sem):
    c = lax.axis_index("c")
    s = lax.axis_index("s")
    wid = s * 2 + c
    base = wid * RPW
    pltpu.sync_copy(xidx_hbm.at[pl.ds(base, RPW)], idx_v)
    pltpu.sync_copy(dst_hbm.at[pl.ds(wid * EPW, EPW)], dst_v)
    pltpu.sync_copy(attr_hbm.at[pl.ds(wid * EPW, EPW)], attr_v)
    descs = []
    for ch in range(4):
        descs.append(pltpu.async_copy(
            emb_hbm.at[idx_v.at[pl.ds(ch * 80, 80)]],
            rows_v.at[pl.ds(ch * 80, 80)], sem))
    zz = jnp.zeros((L,), jnp.float32)

    def zbody(i, _):
        deg_v[pl.ds(i * L, L)] = zz
        return 0
    lax.fori_loop(0, NPAD // L, zbody, 0)

    def dbody(g, _):
        ids = dst_v[pl.ds(g * L, L)]
        vals = attr_v[pl.ds(g * L, L)]
        plsc.addupdate_scatter(deg_v, [ids], vals)
        return 0
    lax.fori_loop(0, NGA, dbody, 0)
    pltpu.sync_copy(deg_v, degp_hbm.at[pl.ds(wid * NPAD, NPAD)])

    for dsc in descs:
        dsc.wait()
    pltpu.sync_copy(rows_v, h0_hbm.at[pl.ds(base, RPW)])


@functools.partial(
    pl.kernel,
    out_type=jax.ShapeDtypeStruct((2, NPAD, D), jnp.float32),
    mesh=_mesh(),
    scratch_types=[
        pltpu.VMEM((2, KEB), jnp.int32),
        pltpu.VMEM((2, KEB), jnp.int32),
        pltpu.VMEM((2, KEB), jnp.float32),
        pltpu.VMEM((KEB, D), jnp.float32),
        pltpu.VMEM((KEB, D), jnp.float32),
        pltpu.VMEM_SHARED((NPAD, D), jnp.float32),
        pltpu.SemaphoreType.DMA,
        pltpu.SemaphoreType.DMA,
        pltpu.SemaphoreType.DMA,
        pltpu.SemaphoreType.DMA,
    ],
    compiler_params=_SC_PARAMS,
)
def _edge_pass(hls_hbm, src_hbm, dst_hbm, attr_hbm, acc_hbm,
               srcw, dstw, attrw, buf0, buf1, acc_sh,
               semw0, semw1, semr0, semr1):
    c = lax.axis_index("c")
    s = lax.axis_index("s")
    wid = s * 2 + c
    bufs = (buf0, buf1)
    semw = (semw0, semw1)
    semr = (semr0, semr1)

    zz = jnp.zeros((L,), jnp.float32)

    def zbody(r, _):
        for cc in range(D // L):
            buf0[r, pl.ds(cc * L, L)] = zz
        return 0
    lax.fori_loop(0, KEB, zbody, 0)
    for half in range(RPS // KEB):
        pltpu.sync_copy(buf0, acc_sh.at[pl.ds(s * RPS + half * KEB, KEB)])
    plsc.subcore_barrier()

    def stage(g, p):
        eo = pl.ds(wid * EPW + g * KEB, KEB)
        pltpu.async_copy(src_hbm.at[eo], srcw.at[p], semw[p])
        pltpu.async_copy(dst_hbm.at[eo], dstw.at[p], semw[p])
        pltpu.async_copy(attr_hbm.at[eo], attrw.at[p], semw[p])

    def stage_wait(g, p):
        eo = pl.ds(wid * EPW + g * KEB, KEB)
        pltpu.make_async_copy(src_hbm.at[eo], srcw.at[p], semw[p]).wait()
        pltpu.make_async_copy(dst_hbm.at[eo], dstw.at[p], semw[p]).wait()
        pltpu.make_async_copy(attr_hbm.at[eo], attrw.at[p], semw[p]).wait()

    def fire(p):
        pltpu.async_copy(hls_hbm.at[srcw.at[p]], bufs[p], semr[p])

    def proc(p):
        pltpu.make_async_copy(hls_hbm.at[srcw.at[p]], bufs[p], semr[p]).wait()

        def jb(k, _):
            av = attrw[p, pl.ds(k * L, L)]
            for jj in range(L):
                n = av[jj]
                row = k * L + jj
                for cc in range(D // L):
                    sl = pl.ds(cc * L, L)
                    bufs[p][row, sl] = n * bufs[p][row, sl]
            return 0
        lax.fori_loop(0, KEB // L, jb, 0)
        pltpu.sync_copy(bufs[p], acc_sh.at[dstw.at[p]], add=True)

    stage(0, 0)
    stage_wait(0, 0)
    fire(0)
    stage(1, 1)

    def body(g2, _):
        for b in range(2):
            g = g2 * 2 + b
            stage_wait(g + 1, 1 - b)
            fire(1 - b)
            proc(b)

            @pl.when(g + 2 < NGB)
            def _():
                stage(g + 2, b)
        return 0
    lax.fori_loop(0, (NGB - 1) // 2, body, 0)
    proc(0)

    plsc.subcore_barrier()
    for half in range(RPS // KEB):
        off = s * RPS + half * KEB
        pltpu.sync_copy(acc_sh.at[pl.ds(off, KEB)], buf0)
        pltpu.sync_copy(buf0, acc_hbm.at[c, pl.ds(off, KEB)])


def _bn_linear(h, gamma, beta, W, dis):
    rowmask = lax.broadcasted_iota(jnp.int32, (NPAD, 1), 0) < N
    hm = jnp.where(rowmask, h, 0.0)
    mean = jnp.sum(hm, axis=0) / N
    dev = jnp.where(rowmask, h - mean, 0.0)
    var = jnp.sum(dev * dev, axis=0) / N
    hn = (h - mean) * lax.rsqrt(var + 1e-5) * gamma + beta
    hl = lax.dot_general(hn, W, (((1,), (1,)), ((), ())),
                         precision=lax.Precision.HIGHEST)
    return dis[:, None] * hl


def _tc1_body(h0_ref, degp_ref, g0_ref, be0_ref, W0_ref, dis_ref, hls_ref):
    deg = jnp.sum(degp_ref[...], axis=0)
    dis = jnp.where(deg > 0, lax.rsqrt(jnp.maximum(deg, 1e-12)), 0.0)
    dis_ref[...] = dis
    hls_ref[...] = _bn_linear(h0_ref[...], g0_ref[...], be0_ref[...],
                              W0_ref[...], dis)


_tc1 = pl.pallas_call(
    _tc1_body,
    out_shape=(
        jax.ShapeDtypeStruct((NPAD,), jnp.float32),
        jax.ShapeDtypeStruct((NPAD, D), jnp.float32),
    ),
)


def _tc2_body(acc_ref, dis_ref, b0_ref, g1_ref, be1_ref, W1_ref, hls_ref):
    dis = dis_ref[...]
    h = jnp.maximum(dis[:, None] * (acc_ref[0] + acc_ref[1]) + b0_ref[...],
                    0.0)
    hls_ref[...] = _bn_linear(h, g1_ref[...], be1_ref[...], W1_ref[...], dis)


_tc2 = pl.pallas_call(
    _tc2_body,
    out_shape=jax.ShapeDtypeStruct((NPAD, D), jnp.float32),
)


def _tc3_body(acc_ref, dis_ref, b1_ref, tf_ref, batch_ref, out_ref):
    dis = dis_ref[...]
    h = jnp.maximum(dis[:, None] * (acc_ref[0] + acc_ref[1]) + b1_ref[...],
                    0.0)
    batch = batch_ref[...]
    tf2 = tf_ref[...][:, None]
    gids = lax.broadcasted_iota(jnp.int32, (NPAD, G), 1)
    m = (batch[:, None] == gids).astype(jnp.float32)
    ex = jnp.exp(tf2)
    num = lax.dot_general(m, ex * h, (((0,), (0,)), ((), ())),
                          precision=lax.Precision.HIGHEST)
    den = lax.dot_general(m, ex, (((0,), (0,)), ((), ())),
                          precision=lax.Precision.HIGHEST)
    out_ref[...] = num / (den + 1e-16)


_tc3 = pl.pallas_call(
    _tc3_body,
    out_shape=jax.ShapeDtypeStruct((G, D), jnp.float32),
)


def kernel(x, edge_index, batch, edge_attr, emb_table,
           bn_gamma0, bn_beta0, W0, b0, bn_gamma1, bn_beta1, W1, b1):
    x_idx = x[:, 0].astype(jnp.int32)
    tfidf = x[:, 1]
    xp = jnp.pad(x_idx, (0, NPAD - N))
    src = edge_index[0].astype(jnp.int32)
    dst = edge_index[1].astype(jnp.int32)

    h0, degp = _emb_deg(xp, dst, edge_attr, emb_table)
    dis, hls0 = _tc1(h0, degp.reshape(NW, NPAD), bn_gamma0, bn_beta0, W0)
    acc0 = _edge_pass(hls0, src, dst, edge_attr)
    hls1 = _tc2(acc0, dis, b0, bn_gamma1, bn_beta1, W1)
    acc1 = _edge_pass(hls1, src, dst, edge_attr)

    batch_p = jnp.pad(batch.astype(jnp.int32), (0, NPAD - N),
                      constant_values=G)
    tf_p = jnp.pad(tfidf, (0, NPAD - N))
    return _tc3(acc1, dis, b1, tf_p, batch_p)


# speedup vs baseline: 19.8807x; 1.1759x over previous
import functools

import jax
import jax.numpy as jnp
from jax import lax
from jax.experimental import pallas as pl
from jax.experimental.pallas import tpu as pltpu
from jax.experimental.pallas import tpu_sc as plsc

N = 10000
E = 320000
V = 100000
D = 128
G = 256

NPAD = 10240
NW = 32
RPW = NPAD // NW
RPS = NPAD // 16
EPW = E // NW
L = 16
NGA = EPW // L
KEB = 80
NGB = EPW // KEB


def _mesh():
    return plsc.VectorSubcoreMesh(core_axis_name="c", subcore_axis_name="s")


_SC_PARAMS = pltpu.CompilerParams(needs_layout_passes=False)


@functools.partial(
    pl.kernel,
    out_type=(
        jax.ShapeDtypeStruct((NPAD, D), jnp.float32),
        jax.ShapeDtypeStruct((NW * NPAD,), jnp.float32),
    ),
    mesh=_mesh(),
    scratch_types=[
        pltpu.VMEM((RPW,), jnp.int32),
        pltpu.VMEM((RPW, D), jnp.float32),
        pltpu.VMEM((EPW,), jnp.int32),
        pltpu.VMEM((EPW,), jnp.float32),
        pltpu.VMEM((NPAD,), jnp.float32),
        pltpu.SemaphoreType.DMA,
    ],
    compiler_params=_SC_PARAMS,
)
def _emb_deg(xidx_hbm, dst_hbm, attr_hbm, emb_hbm, h0_hbm, degp_hbm,
             idx_v, rows_v, dst_v, attr_v, deg_v, sem):
    c = lax.axis_index("c")
    s = lax.axis_index("s")
    wid = s * 2 + c
    base = wid * RPW
    pltpu.sync_copy(xidx_hbm.at[pl.ds(base, RPW)], idx_v)
    pltpu.sync_copy(dst_hbm.at[pl.ds(wid * EPW, EPW)], dst_v)
    pltpu.sync_copy(attr_hbm.at[pl.ds(wid * EPW, EPW)], attr_v)
    descs = []
    for ch in range(4):
        descs.append(pltpu.async_copy(
            emb_hbm.at[idx_v.at[pl.ds(ch * 80, 80)]],
            rows_v.at[pl.ds(ch * 80, 80)], sem))
    zz = jnp.zeros((L,), jnp.float32)

    def zbody(i, _):
        deg_v[pl.ds(i * L, L)] = zz
        return 0
    lax.fori_loop(0, NPAD // L, zbody, 0)

    def dbody(g, _):
        ids = dst_v[pl.ds(g * L, L)]
        vals = attr_v[pl.ds(g * L, L)]
        plsc.addupdate_scatter(deg_v, [ids], vals)
        return 0
    lax.fori_loop(0, NGA, dbody, 0)
    pltpu.sync_copy(deg_v, degp_hbm.at[pl.ds(wid * NPAD, NPAD)])

    for dsc in descs:
        dsc.wait()
    pltpu.sync_copy(rows_v, h0_hbm.at[pl.ds(base, RPW)])


@functools.partial(
    pl.kernel,
    out_type=jax.ShapeDtypeStruct((2, NPAD, D), jnp.float32),
    mesh=_mesh(),
    scratch_types=[
        pltpu.VMEM((2, KEB), jnp.int32),
        pltpu.VMEM((2, KEB), jnp.int32),
        pltpu.VMEM((2, KEB), jnp.float32),
        pltpu.VMEM((2, KEB), jnp.int32),
        pltpu.VMEM((KEB, D), jnp.float32),
        pltpu.VMEM((KEB, D), jnp.float32),
        pltpu.VMEM_SHARED((NPAD, D), jnp.float32),
        pltpu.SemaphoreType.DMA,
        pltpu.SemaphoreType.DMA,
        pltpu.SemaphoreType.DMA,
        pltpu.SemaphoreType.DMA,
        pltpu.SemaphoreType.DMA,
        pltpu.SemaphoreType.DMA,
    ],
    compiler_params=_SC_PARAMS,
)
def _edge_pass(hls_hbm, src_hbm, dst_hbm, attr_hbm, acc_hbm,
               srcw, dstw, attrw, dsta, buf0, buf1, acc_sh,
               semw0, semw1, semr0, semr1, sems0, sems1):
    c = lax.axis_index("c")
    s = lax.axis_index("s")
    wid = s * 2 + c
    bufs = (buf0, buf1)
    semw = (semw0, semw1)
    semr = (semr0, semr1)
    sems = (sems0, sems1)

    zz = jnp.zeros((L,), jnp.float32)

    def zbody(r, _):
        for cc in range(D // L):
            buf0[r, pl.ds(cc * L, L)] = zz
        return 0
    lax.fori_loop(0, KEB, zbody, 0)
    for half in range(RPS // KEB):
        pltpu.sync_copy(buf0, acc_sh.at[pl.ds(s * RPS + half * KEB, KEB)])
    plsc.subcore_barrier()

    def stage(g, p):
        eo = pl.ds(wid * EPW + g * KEB, KEB)
        pltpu.async_copy(src_hbm.at[eo], srcw.at[p], semw[p])
        pltpu.async_copy(dst_hbm.at[eo], dstw.at[p], semw[p])
        pltpu.async_copy(attr_hbm.at[eo], attrw.at[p], semw[p])

    def stage_wait(g, p):
        eo = pl.ds(wid * EPW + g * KEB, KEB)
        pltpu.make_async_copy(src_hbm.at[eo], srcw.at[p], semw[p]).wait()
        pltpu.make_async_copy(dst_hbm.at[eo], dstw.at[p], semw[p]).wait()
        pltpu.make_async_copy(attr_hbm.at[eo], attrw.at[p], semw[p]).wait()

    def fire(p):
        pltpu.async_copy(hls_hbm.at[srcw.at[p]], bufs[p], semr[p])

    def scat_wait(p):
        pltpu.make_async_copy(bufs[p], acc_sh.at[dsta.at[p]], sems[p]).wait()

    def proc(p):
        pltpu.make_async_copy(hls_hbm.at[srcw.at[p]], bufs[p], semr[p]).wait()

        def jb(k, _):
            av = attrw[p, pl.ds(k * L, L)]
            for jj in range(L):
                n = av[jj]
                row = k * L + jj
                for cc in range(D // L):
                    sl = pl.ds(cc * L, L)
                    bufs[p][row, sl] = n * bufs[p][row, sl]
            return 0
        lax.fori_loop(0, KEB // L, jb, 0)
        for k5 in range(KEB // L):
            dsta[p, pl.ds(k5 * L, L)] = dstw[p, pl.ds(k5 * L, L)]
        pltpu.async_copy(bufs[p], acc_sh.at[dsta.at[p]], sems[p], add=True)

    stage(0, 0)
    stage_wait(0, 0)
    fire(0)
    stage(1, 1)

    def body(g2, _):
        for b in range(2):
            g = g2 * 2 + b
            stage_wait(g + 1, 1 - b)

            @pl.when(g >= 1)
            def _():
                scat_wait(1 - b)
            fire(1 - b)
            proc(b)

            @pl.when(g + 2 < NGB)
            def _():
                stage(g + 2, b)
        return 0
    lax.fori_loop(0, (NGB - 1) // 2, body, 0)
    scat_wait(1)
    proc(0)
    scat_wait(0)

    plsc.subcore_barrier()
    for half in range(RPS // KEB):
        off = s * RPS + half * KEB
        pltpu.sync_copy(acc_sh.at[pl.ds(off, KEB)], buf0)
        pltpu.sync_copy(buf0, acc_hbm.at[c, pl.ds(off, KEB)])


def _bn_linear(h, gamma, beta, W, dis):
    rowmask = lax.broadcasted_iota(jnp.int32, (NPAD, 1), 0) < N
    hm = jnp.where(rowmask, h, 0.0)
    mean = jnp.sum(hm, axis=0) / N
    dev = jnp.where(rowmask, h - mean, 0.0)
    var = jnp.sum(dev * dev, axis=0) / N
    hn = (h - mean) * lax.rsqrt(var + 1e-5) * gamma + beta
    hl = lax.dot_general(hn, W, (((1,), (1,)), ((), ())),
                         precision=lax.Precision.HIGHEST)
    return dis[:, None] * hl


def _tc1_body(h0_ref, degp_ref, g0_ref, be0_ref, W0_ref, dis_ref, hls_ref):
    deg = jnp.sum(degp_ref[...], axis=0)
    dis = jnp.where(deg > 0, lax.rsqrt(jnp.maximum(deg, 1e-12)), 0.0)
    dis_ref[...] = dis
    hls_ref[...] = _bn_linear(h0_ref[...], g0_ref[...], be0_ref[...],
                              W0_ref[...], dis)


_tc1 = pl.pallas_call(
    _tc1_body,
    out_shape=(
        jax.ShapeDtypeStruct((NPAD,), jnp.float32),
        jax.ShapeDtypeStruct((NPAD, D), jnp.float32),
    ),
)


def _tc2_body(acc_ref, dis_ref, b0_ref, g1_ref, be1_ref, W1_ref, hls_ref):
    dis = dis_ref[...]
    h = jnp.maximum(dis[:, None] * (acc_ref[0] + acc_ref[1]) + b0_ref[...],
                    0.0)
    hls_ref[...] = _bn_linear(h, g1_ref[...], be1_ref[...], W1_ref[...], dis)


_tc2 = pl.pallas_call(
    _tc2_body,
    out_shape=jax.ShapeDtypeStruct((NPAD, D), jnp.float32),
)


def _tc3_body(acc_ref, dis_ref, b1_ref, tf_ref, batch_ref, out_ref):
    dis = dis_ref[...]
    h = jnp.maximum(dis[:, None] * (acc_ref[0] + acc_ref[1]) + b1_ref[...],
                    0.0)
    batch = batch_ref[...]
    tf2 = tf_ref[...][:, None]
    gids = lax.broadcasted_iota(jnp.int32, (NPAD, G), 1)
    m = (batch[:, None] == gids).astype(jnp.float32)
    ex = jnp.exp(tf2)
    num = lax.dot_general(m, ex * h, (((0,), (0,)), ((), ())),
                          precision=lax.Precision.HIGHEST)
    den = lax.dot_general(m, ex, (((0,), (0,)), ((), ())),
                          precision=lax.Precision.HIGHEST)
    out_ref[...] = num / (den + 1e-16)


_tc3 = pl.pallas_call(
    _tc3_body,
    out_shape=jax.ShapeDtypeStruct((G, D), jnp.float32),
)


def kernel(x, edge_index, batch, edge_attr, emb_table,
           bn_gamma0, bn_beta0, W0, b0, bn_gamma1, bn_beta1, W1, b1):
    x_idx = x[:, 0].astype(jnp.int32)
    tfidf = x[:, 1]
    xp = jnp.pad(x_idx, (0, NPAD - N))
    src = edge_index[0].astype(jnp.int32)
    dst = edge_index[1].astype(jnp.int32)

    h0, degp = _emb_deg(xp, dst, edge_attr, emb_table)
    dis, hls0 = _tc1(h0, degp.reshape(NW, NPAD), bn_gamma0, bn_beta0, W0)
    acc0 = _edge_pass(hls0, src, dst, edge_attr)
    hls1 = _tc2(acc0, dis, b0, bn_gamma1, bn_beta1, W1)
    acc1 = _edge_pass(hls1, src, dst, edge_attr)

    batch_p = jnp.pad(batch.astype(jnp.int32), (0, NPAD - N),
                      constant_values=G)
    tf_p = jnp.pad(tfidf, (0, NPAD - N))
    return _tc3(acc1, dis, b1, tf_p, batch_p)


# speedup vs baseline: 19.9674x; 1.0044x over previous
import functools

import jax
import jax.numpy as jnp
from jax import lax
from jax.experimental import pallas as pl
from jax.experimental.pallas import tpu as pltpu
from jax.experimental.pallas import tpu_sc as plsc

N = 10000
E = 320000
V = 100000
D = 128
G = 256

NPAD = 10240
NW = 32
RPW = NPAD // NW
RPS = NPAD // 16
EPW = E // NW
L = 16
NGA = EPW // L
KEB = 80
NGB = EPW // KEB


def _mesh():
    return plsc.VectorSubcoreMesh(core_axis_name="c", subcore_axis_name="s")


_SC_PARAMS = pltpu.CompilerParams(needs_layout_passes=False)


@functools.partial(
    pl.kernel,
    out_type=(
        jax.ShapeDtypeStruct((NPAD, D), jnp.float32),
        jax.ShapeDtypeStruct((NW * NPAD,), jnp.float32),
    ),
    mesh=_mesh(),
    scratch_types=[
        pltpu.VMEM((RPW,), jnp.int32),
        pltpu.VMEM((RPW, D), jnp.float32),
        pltpu.VMEM((EPW,), jnp.int32),
        pltpu.VMEM((EPW,), jnp.float32),
        pltpu.VMEM((NPAD,), jnp.float32),
        pltpu.SemaphoreType.DMA,
    ],
    compiler_params=_SC_PARAMS,
)
def _emb_deg(xidx_hbm, dst_hbm, attr_hbm, emb_hbm, h0_hbm, degp_hbm,
             idx_v, rows_v, dst_v, attr_v, deg_v, sem):
    c = lax.axis_index("c")
    s = lax.axis_index("s")
    wid = s * 2 + c
    base = wid * RPW
    pltpu.sync_copy(xidx_hbm.at[pl.ds(base, RPW)], idx_v)
    pltpu.sync_copy(dst_hbm.at[pl.ds(wid * EPW, EPW)], dst_v)
    pltpu.sync_copy(attr_hbm.at[pl.ds(wid * EPW, EPW)], attr_v)
    descs = []
    for ch in range(4):
        descs.append(pltpu.async_copy(
            emb_hbm.at[idx_v.at[pl.ds(ch * 80, 80)]],
            rows_v.at[pl.ds(ch * 80, 80)], sem))
    zz = jnp.zeros((L,), jnp.float32)

    def zbody(i, _):
        deg_v[pl.ds(i * L, L)] = zz
        return 0
    lax.fori_loop(0, NPAD // L, zbody, 0)

    def dbody(g, _):
        ids = dst_v[pl.ds(g * L, L)]
        vals = attr_v[pl.ds(g * L, L)]
        plsc.addupdate_scatter(deg_v, [ids], vals)
        return 0
    lax.fori_loop(0, NGA, dbody, 0)
    pltpu.sync_copy(deg_v, degp_hbm.at[pl.ds(wid * NPAD, NPAD)])

    for dsc in descs:
        dsc.wait()
    pltpu.sync_copy(rows_v, h0_hbm.at[pl.ds(base, RPW)])


@functools.partial(
    pl.kernel,
    out_type=jax.ShapeDtypeStruct((2, NPAD, D), jnp.float32),
    mesh=_mesh(),
    scratch_types=[
        pltpu.VMEM((2, KEB), jnp.int32),
        pltpu.VMEM((2, KEB), jnp.int32),
        pltpu.VMEM((2, KEB), jnp.float32),
        pltpu.VMEM((2, KEB), jnp.int32),
        pltpu.VMEM((KEB, D), jnp.float32),
        pltpu.VMEM((KEB, D), jnp.float32),
        pltpu.VMEM_SHARED((NPAD, D), jnp.float32),
        pltpu.SemaphoreType.DMA,
        pltpu.SemaphoreType.DMA,
        pltpu.SemaphoreType.DMA,
        pltpu.SemaphoreType.DMA,
        pltpu.SemaphoreType.DMA,
        pltpu.SemaphoreType.DMA,
    ],
    compiler_params=_SC_PARAMS,
)
def _edge_pass(hls_hbm, src_hbm, dst_hbm, attr_hbm, acc_hbm,
               srcw, dstw, attrw, dsta, buf0, buf1, acc_sh,
               semw0, semw1, semr0, semr1, sems0, sems1):
    c = lax.axis_index("c")
    s = lax.axis_index("s")
    wid = s * 2 + c
    bufs = (buf0, buf1)
    semw = (semw0, semw1)
    semr = (semr0, semr1)
    sems = (sems0, sems1)

    zz = jnp.zeros((L,), jnp.float32)

    def zbody(r, _):
        for cc in range(D // L):
            buf0[r, pl.ds(cc * L, L)] = zz
        return 0
    lax.fori_loop(0, KEB, zbody, 0)
    for half in range(RPS // KEB):
        pltpu.sync_copy(buf0, acc_sh.at[pl.ds(s * RPS + half * KEB, KEB)])
    plsc.subcore_barrier()

    def stage(g, p):
        eo = pl.ds(wid * EPW + g * KEB, KEB)
        pltpu.async_copy(src_hbm.at[eo], srcw.at[p], semw[p])
        pltpu.async_copy(dst_hbm.at[eo], dstw.at[p], semw[p])
        pltpu.async_copy(attr_hbm.at[eo], attrw.at[p], semw[p])

    def stage_wait(g, p):
        eo = pl.ds(wid * EPW + g * KEB, KEB)
        pltpu.make_async_copy(src_hbm.at[eo], srcw.at[p], semw[p]).wait()
        pltpu.make_async_copy(dst_hbm.at[eo], dstw.at[p], semw[p]).wait()
        pltpu.make_async_copy(attr_hbm.at[eo], attrw.at[p], semw[p]).wait()

    def fire(p):
        pltpu.async_copy(hls_hbm.at[srcw.at[p]], bufs[p], semr[p])

    def scat_wait(p):
        pltpu.make_async_copy(bufs[p], acc_sh.at[dsta.at[p]], sems[p]).wait()

    def proc(p):
        pltpu.make_async_copy(hls_hbm.at[srcw.at[p]], bufs[p], semr[p]).wait()

        def jb(k, _):
            av = attrw[p, pl.ds(k * L, L)]
            for jj in range(L):
                n = av[jj]
                row = k * L + jj
                for cc in range(D // L):
                    sl = pl.ds(cc * L, L)
                    bufs[p][row, sl] = n * bufs[p][row, sl]
            return 0
        lax.fori_loop(0, KEB // L, jb, 0)
        for k5 in range(KEB // L):
            dsta[p, pl.ds(k5 * L, L)] = dstw[p, pl.ds(k5 * L, L)]
        pltpu.async_copy(bufs[p], acc_sh.at[dsta.at[p]], sems[p], add=True)

    stage(0, 0)
    stage_wait(0, 0)
    fire(0)
    stage(1, 1)

    def body(g2, _):
        for b in range(2):
            g = g2 * 2 + b
            stage_wait(g + 1, 1 - b)

            @pl.when(g >= 1)
            def _():
                scat_wait(1 - b)
            fire(1 - b)
            proc(b)

            @pl.when(g + 2 < NGB)
            def _():
                stage(g + 2, b)
        return 0
    lax.fori_loop(0, (NGB - 1) // 2, body, 0)
    scat_wait(1)
    proc(0)
    scat_wait(0)

    plsc.subcore_barrier()
    pltpu.sync_copy(acc_sh.at[pl.ds(s * RPS, RPS)],
                    acc_hbm.at[c, pl.ds(s * RPS, RPS)])


def _bn_linear(h, gamma, beta, W, dis):
    rowmask = lax.broadcasted_iota(jnp.int32, (NPAD, 1), 0) < N
    hm = jnp.where(rowmask, h, 0.0)
    mean = jnp.sum(hm, axis=0) / N
    dev = jnp.where(rowmask, h - mean, 0.0)
    var = jnp.sum(dev * dev, axis=0) / N
    hn = (h - mean) * lax.rsqrt(var + 1e-5) * gamma + beta
    hl = lax.dot_general(hn, W, (((1,), (1,)), ((), ())),
                         precision=lax.Precision.HIGHEST)
    return dis[:, None] * hl


def _tc1_body(h0_ref, degp_ref, g0_ref, be0_ref, W0_ref, dis_ref, hls_ref):
    deg = jnp.sum(degp_ref[...], axis=0)
    dis = jnp.where(deg > 0, lax.rsqrt(jnp.maximum(deg, 1e-12)), 0.0)
    dis_ref[...] = dis
    hls_ref[...] = _bn_linear(h0_ref[...], g0_ref[...], be0_ref[...],
                              W0_ref[...], dis)


_tc1 = pl.pallas_call(
    _tc1_body,
    out_shape=(
        jax.ShapeDtypeStruct((NPAD,), jnp.float32),
        jax.ShapeDtypeStruct((NPAD, D), jnp.float32),
    ),
)


def _tc2_body(acc_ref, dis_ref, b0_ref, g1_ref, be1_ref, W1_ref, hls_ref):
    dis = dis_ref[...]
    h = jnp.maximum(dis[:, None] * (acc_ref[0] + acc_ref[1]) + b0_ref[...],
                    0.0)
    hls_ref[...] = _bn_linear(h, g1_ref[...], be1_ref[...], W1_ref[...], dis)


_tc2 = pl.pallas_call(
    _tc2_body,
    out_shape=jax.ShapeDtypeStruct((NPAD, D), jnp.float32),
)


def _tc3_body(acc_ref, dis_ref, b1_ref, tf_ref, batch_ref, out_ref):
    dis = dis_ref[...]
    h = jnp.maximum(dis[:, None] * (acc_ref[0] + acc_ref[1]) + b1_ref[...],
                    0.0)
    batch = batch_ref[...]
    tf2 = tf_ref[...][:, None]
    gids = lax.broadcasted_iota(jnp.int32, (NPAD, G), 1)
    m = (batch[:, None] == gids).astype(jnp.float32)
    ex = jnp.exp(tf2)
    num = lax.dot_general(m, ex * h, (((0,), (0,)), ((), ())),
                          precision=lax.Precision.HIGHEST)
    den = lax.dot_general(m, ex, (((0,), (0,)), ((), ())),
                          precision=lax.Precision.HIGHEST)
    out_ref[...] = num / (den + 1e-16)


_tc3 = pl.pallas_call(
    _tc3_body,
    out_shape=jax.ShapeDtypeStruct((G, D), jnp.float32),
)


def kernel(x, edge_index, batch, edge_attr, emb_table,
           bn_gamma0, bn_beta0, W0, b0, bn_gamma1, bn_beta1, W1, b1):
    x_idx = x[:, 0].astype(jnp.int32)
    tfidf = x[:, 1]
    xp = jnp.pad(x_idx, (0, NPAD - N))
    src = edge_index[0].astype(jnp.int32)
    dst = edge_index[1].astype(jnp.int32)

    h0, degp = _emb_deg(xp, dst, edge_attr, emb_table)
    dis, hls0 = _tc1(h0, degp.reshape(NW, NPAD), bn_gamma0, bn_beta0, W0)
    acc0 = _edge_pass(hls0, src, dst, edge_attr)
    hls1 = _tc2(acc0, dis, b0, bn_gamma1, bn_beta1, W1)
    acc1 = _edge_pass(hls1, src, dst, edge_attr)

    batch_p = jnp.pad(batch.astype(jnp.int32), (0, NPAD - N),
                      constant_values=G)
    tf_p = jnp.pad(tfidf, (0, NPAD - N))
    return _tc3(acc1, dis, b1, tf_p, batch_p)


# speedup vs baseline: 22.5627x; 1.1300x over previous
import functools

import jax
import jax.numpy as jnp
from jax import lax
from jax.experimental import pallas as pl
from jax.experimental.pallas import tpu as pltpu
from jax.experimental.pallas import tpu_sc as plsc

N = 10000
E = 320000
V = 100000
D = 128
G = 256

NPAD = 10240
NW = 32
RPW = NPAD // NW
RPS = NPAD // 16
EPW = E // NW
L = 16
NGA = EPW // L
KEB = 80
NGB = EPW // KEB
NB = 4


def _mesh():
    return plsc.VectorSubcoreMesh(core_axis_name="c", subcore_axis_name="s")


_SC_PARAMS = pltpu.CompilerParams(needs_layout_passes=False)


@functools.partial(
    pl.kernel,
    out_type=(
        jax.ShapeDtypeStruct((NPAD, D), jnp.float32),
        jax.ShapeDtypeStruct((NW * NPAD,), jnp.float32),
    ),
    mesh=_mesh(),
    scratch_types=[
        pltpu.VMEM((RPW,), jnp.int32),
        pltpu.VMEM((RPW, D), jnp.float32),
        pltpu.VMEM((EPW,), jnp.int32),
        pltpu.VMEM((EPW,), jnp.float32),
        pltpu.VMEM((NPAD,), jnp.float32),
        pltpu.SemaphoreType.DMA,
    ],
    compiler_params=_SC_PARAMS,
)
def _emb_deg(xidx_hbm, dst_hbm, attr_hbm, emb_hbm, h0_hbm, degp_hbm,
             idx_v, rows_v, dst_v, attr_v, deg_v, sem):
    c = lax.axis_index("c")
    s = lax.axis_index("s")
    wid = s * 2 + c
    base = wid * RPW
    pltpu.sync_copy(xidx_hbm.at[pl.ds(base, RPW)], idx_v)
    pltpu.sync_copy(dst_hbm.at[pl.ds(wid * EPW, EPW)], dst_v)
    pltpu.sync_copy(attr_hbm.at[pl.ds(wid * EPW, EPW)], attr_v)
    descs = []
    for ch in range(4):
        descs.append(pltpu.async_copy(
            emb_hbm.at[idx_v.at[pl.ds(ch * 80, 80)]],
            rows_v.at[pl.ds(ch * 80, 80)], sem))
    zz = jnp.zeros((L,), jnp.float32)

    def zbody(i, _):
        deg_v[pl.ds(i * L, L)] = zz
        return 0
    lax.fori_loop(0, NPAD // L, zbody, 0)

    def dbody(g, _):
        ids = dst_v[pl.ds(g * L, L)]
        vals = attr_v[pl.ds(g * L, L)]
        plsc.addupdate_scatter(deg_v, [ids], vals)
        return 0
    lax.fori_loop(0, NGA, dbody, 0)
    pltpu.sync_copy(deg_v, degp_hbm.at[pl.ds(wid * NPAD, NPAD)])

    for dsc in descs:
        dsc.wait()
    pltpu.sync_copy(rows_v, h0_hbm.at[pl.ds(base, RPW)])


@functools.partial(
    pl.kernel,
    out_type=jax.ShapeDtypeStruct((2, NPAD, D), jnp.float32),
    mesh=_mesh(),
    scratch_types=[
        pltpu.VMEM((NB, KEB), jnp.int32),
        pltpu.VMEM((NB, KEB), jnp.int32),
        pltpu.VMEM((NB, KEB), jnp.float32),
        pltpu.VMEM((NB, KEB), jnp.int32),
        pltpu.VMEM((KEB, D), jnp.float32),
        pltpu.VMEM((KEB, D), jnp.float32),
        pltpu.VMEM((KEB, D), jnp.float32),
        pltpu.VMEM((KEB, D), jnp.float32),
        pltpu.VMEM_SHARED((NPAD, D), jnp.float32),
        pltpu.SemaphoreType.DMA,
        pltpu.SemaphoreType.DMA,
        pltpu.SemaphoreType.DMA,
        pltpu.SemaphoreType.DMA,
        pltpu.SemaphoreType.DMA,
        pltpu.SemaphoreType.DMA,
        pltpu.SemaphoreType.DMA,
        pltpu.SemaphoreType.DMA,
        pltpu.SemaphoreType.DMA,
        pltpu.SemaphoreType.DMA,
        pltpu.SemaphoreType.DMA,
        pltpu.SemaphoreType.DMA,
    ],
    compiler_params=_SC_PARAMS,
)
def _edge_pass(hls_hbm, src_hbm, dst_hbm, attr_hbm, acc_hbm,
               srcw, dstw, attrw, dsta, buf0, buf1, buf2, buf3, acc_sh,
               semw0, semw1, semw2, semw3, semr0, semr1, semr2, semr3,
               sems0, sems1, sems2, sems3):
    c = lax.axis_index("c")
    s = lax.axis_index("s")
    wid = s * 2 + c
    bufs = (buf0, buf1, buf2, buf3)
    semw = (semw0, semw1, semw2, semw3)
    semr = (semr0, semr1, semr2, semr3)
    sems = (sems0, sems1, sems2, sems3)

    zz = jnp.zeros((L,), jnp.float32)

    def zbody(r, _):
        for cc in range(D // L):
            buf0[r, pl.ds(cc * L, L)] = zz
        return 0
    lax.fori_loop(0, KEB, zbody, 0)
    for half in range(RPS // KEB):
        pltpu.sync_copy(buf0, acc_sh.at[pl.ds(s * RPS + half * KEB, KEB)])
    plsc.subcore_barrier()

    def stage(g, p):
        eo = pl.ds(wid * EPW + g * KEB, KEB)
        pltpu.async_copy(src_hbm.at[eo], srcw.at[p], semw[p])
        pltpu.async_copy(dst_hbm.at[eo], dstw.at[p], semw[p])
        pltpu.async_copy(attr_hbm.at[eo], attrw.at[p], semw[p])

    def stage_wait(g, p):
        eo = pl.ds(wid * EPW + g * KEB, KEB)
        pltpu.make_async_copy(src_hbm.at[eo], srcw.at[p], semw[p]).wait()
        pltpu.make_async_copy(dst_hbm.at[eo], dstw.at[p], semw[p]).wait()
        pltpu.make_async_copy(attr_hbm.at[eo], attrw.at[p], semw[p]).wait()

    def fire(p):
        pltpu.async_copy(hls_hbm.at[srcw.at[p]], bufs[p], semr[p])

    def scat_wait(p):
        pltpu.make_async_copy(bufs[p], acc_sh.at[dsta.at[p]], sems[p]).wait()

    def proc(p):
        pltpu.make_async_copy(hls_hbm.at[srcw.at[p]], bufs[p], semr[p]).wait()

        def jb(k, _):
            av = attrw[p, pl.ds(k * L, L)]
            for jj in range(L):
                n = av[jj]
                row = k * L + jj
                for cc in range(D // L):
                    sl = pl.ds(cc * L, L)
                    bufs[p][row, sl] = n * bufs[p][row, sl]
            return 0
        lax.fori_loop(0, KEB // L, jb, 0)
        for k5 in range(KEB // L):
            dsta[p, pl.ds(k5 * L, L)] = dstw[p, pl.ds(k5 * L, L)]
        pltpu.async_copy(bufs[p], acc_sh.at[dsta.at[p]], sems[p], add=True)

    def step(g, ph):
        stage_wait(g + 1, (ph + 1) % NB)

        @pl.when(g >= NB - 1)
        def _():
            scat_wait((ph + 1) % NB)
        fire((ph + 1) % NB)

        @pl.when(g + 2 < NGB)
        def _():
            stage(g + 2, (ph + 2) % NB)
        proc(ph)

    stage(0, 0)
    stage(1, 1)
    stage_wait(0, 0)
    fire(0)

    def body(gq, _):
        for b in range(NB):
            step(gq * NB + b, b)
        return 0
    lax.fori_loop(0, (NGB - 1) // NB, body, 0)
    proc((NGB - 1) % NB)
    for ph in range(NB):
        scat_wait(ph)

    plsc.subcore_barrier()
    pltpu.sync_copy(acc_sh.at[pl.ds(s * RPS, RPS)],
                    acc_hbm.at[c, pl.ds(s * RPS, RPS)])


def _bn_linear(h, gamma, beta, W, dis):
    rowmask = lax.broadcasted_iota(jnp.int32, (NPAD, 1), 0) < N
    hm = jnp.where(rowmask, h, 0.0)
    mean = jnp.sum(hm, axis=0) / N
    dev = jnp.where(rowmask, h - mean, 0.0)
    var = jnp.sum(dev * dev, axis=0) / N
    hn = (h - mean) * lax.rsqrt(var + 1e-5) * gamma + beta
    hl = lax.dot_general(hn, W, (((1,), (1,)), ((), ())),
                         precision=lax.Precision.HIGHEST)
    return dis[:, None] * hl


def _tc1_body(h0_ref, degp_ref, g0_ref, be0_ref, W0_ref, dis_ref, hls_ref):
    deg = jnp.sum(degp_ref[...], axis=0)
    dis = jnp.where(deg > 0, lax.rsqrt(jnp.maximum(deg, 1e-12)), 0.0)
    dis_ref[...] = dis
    hls_ref[...] = _bn_linear(h0_ref[...], g0_ref[...], be0_ref[...],
                              W0_ref[...], dis)


_tc1 = pl.pallas_call(
    _tc1_body,
    out_shape=(
        jax.ShapeDtypeStruct((NPAD,), jnp.float32),
        jax.ShapeDtypeStruct((NPAD, D), jnp.float32),
    ),
)


def _tc2_body(acc_ref, dis_ref, b0_ref, g1_ref, be1_ref, W1_ref, hls_ref):
    dis = dis_ref[...]
    h = jnp.maximum(dis[:, None] * (acc_ref[0] + acc_ref[1]) + b0_ref[...],
                    0.0)
    hls_ref[...] = _bn_linear(h, g1_ref[...], be1_ref[...], W1_ref[...], dis)


_tc2 = pl.pallas_call(
    _tc2_body,
    out_shape=jax.ShapeDtypeStruct((NPAD, D), jnp.float32),
)


def _tc3_body(acc_ref, dis_ref, b1_ref, tf_ref, batch_ref, out_ref):
    dis = dis_ref[...]
    h = jnp.maximum(dis[:, None] * (acc_ref[0] + acc_ref[1]) + b1_ref[...],
                    0.0)
    batch = batch_ref[...]
    tf2 = tf_ref[...][:, None]
    gids = lax.broadcasted_iota(jnp.int32, (NPAD, G), 1)
    m = (batch[:, None] == gids).astype(jnp.float32)
    ex = jnp.exp(tf2)
    num = lax.dot_general(m, ex * h, (((0,), (0,)), ((), ())),
                          precision=lax.Precision.HIGHEST)
    den = lax.dot_general(m, ex, (((0,), (0,)), ((), ())),
                          precision=lax.Precision.HIGHEST)
    out_ref[...] = num / (den + 1e-16)


_tc3 = pl.pallas_call(
    _tc3_body,
    out_shape=jax.ShapeDtypeStruct((G, D), jnp.float32),
)


def kernel(x, edge_index, batch, edge_attr, emb_table,
           bn_gamma0, bn_beta0, W0, b0, bn_gamma1, bn_beta1, W1, b1):
    x_idx = x[:, 0].astype(jnp.int32)
    tfidf = x[:, 1]
    xp = jnp.pad(x_idx, (0, NPAD - N))
    src = edge_index[0].astype(jnp.int32)
    dst = edge_index[1].astype(jnp.int32)

    h0, degp = _emb_deg(xp, dst, edge_attr, emb_table)
    dis, hls0 = _tc1(h0, degp.reshape(NW, NPAD), bn_gamma0, bn_beta0, W0)
    acc0 = _edge_pass(hls0, src, dst, edge_attr)
    hls1 = _tc2(acc0, dis, b0, bn_gamma1, bn_beta1, W1)
    acc1 = _edge_pass(hls1, src, dst, edge_attr)

    batch_p = jnp.pad(batch.astype(jnp.int32), (0, NPAD - N),
                      constant_values=G)
    tf_p = jnp.pad(tfidf, (0, NPAD - N))
    return _tc3(acc1, dis, b1, tf_p, batch_p)


# speedup vs baseline: 23.8549x; 1.0573x over previous
import functools

import jax
import jax.numpy as jnp
from jax import lax
from jax.experimental import pallas as pl
from jax.experimental.pallas import tpu as pltpu
from jax.experimental.pallas import tpu_sc as plsc

N = 10000
E = 320000
V = 100000
D = 128
G = 256

NPAD = 10240
NW = 32
RPW = NPAD // NW
RPS = NPAD // 16
EPW = E // NW
L = 16
NGA = EPW // L
KEB = 80
NGB = EPW // KEB
NB = 4


def _mesh():
    return plsc.VectorSubcoreMesh(core_axis_name="c", subcore_axis_name="s")


_SC_PARAMS = pltpu.CompilerParams(needs_layout_passes=False)


@functools.partial(
    pl.kernel,
    out_type=(
        jax.ShapeDtypeStruct((NPAD, D), jnp.float32),
        jax.ShapeDtypeStruct((NW * NPAD,), jnp.float32),
    ),
    mesh=_mesh(),
    scratch_types=[
        pltpu.VMEM((RPW,), jnp.int32),
        pltpu.VMEM((RPW, D), jnp.float32),
        pltpu.VMEM((EPW,), jnp.int32),
        pltpu.VMEM((EPW,), jnp.float32),
        pltpu.VMEM((NPAD,), jnp.float32),
        pltpu.SemaphoreType.DMA,
    ],
    compiler_params=_SC_PARAMS,
)
def _emb_deg(xidx_hbm, dst_hbm, attr_hbm, emb_hbm, h0_hbm, degp_hbm,
             idx_v, rows_v, dst_v, attr_v, deg_v, sem):
    c = lax.axis_index("c")
    s = lax.axis_index("s")
    wid = s * 2 + c
    base = wid * RPW
    pltpu.sync_copy(xidx_hbm.at[pl.ds(base, RPW)], idx_v)
    pltpu.sync_copy(dst_hbm.at[pl.ds(wid * EPW, EPW)], dst_v)
    pltpu.sync_copy(attr_hbm.at[pl.ds(wid * EPW, EPW)], attr_v)
    descs = []
    for ch in range(4):
        descs.append(pltpu.async_copy(
            emb_hbm.at[idx_v.at[pl.ds(ch * 80, 80)]],
            rows_v.at[pl.ds(ch * 80, 80)], sem))
    zz = jnp.zeros((L,), jnp.float32)

    def zbody(i, _):
        deg_v[pl.ds(i * L, L)] = zz
        return 0
    lax.fori_loop(0, NPAD // L, zbody, 0)

    def dbody(g, _):
        ids = dst_v[pl.ds(g * L, L)]
        vals = attr_v[pl.ds(g * L, L)]
        plsc.addupdate_scatter(deg_v, [ids], vals)
        return 0
    lax.fori_loop(0, NGA, dbody, 0)
    pltpu.sync_copy(deg_v, degp_hbm.at[pl.ds(wid * NPAD, NPAD)])

    for dsc in descs:
        dsc.wait()
    pltpu.sync_copy(rows_v, h0_hbm.at[pl.ds(base, RPW)])


@functools.partial(
    pl.kernel,
    out_type=jax.ShapeDtypeStruct((2, NPAD, D), jnp.float32),
    mesh=_mesh(),
    scratch_types=[
        pltpu.VMEM((NB, KEB), jnp.int32),
        pltpu.VMEM((NB, KEB), jnp.int32),
        pltpu.VMEM((NB, KEB), jnp.float32),
        pltpu.VMEM((NB, KEB), jnp.int32),
        pltpu.VMEM((KEB, D), jnp.float32),
        pltpu.VMEM((KEB, D), jnp.float32),
        pltpu.VMEM((KEB, D), jnp.float32),
        pltpu.VMEM((KEB, D), jnp.float32),
        pltpu.VMEM_SHARED((NPAD, D), jnp.float32),
        pltpu.SemaphoreType.DMA,
        pltpu.SemaphoreType.DMA,
        pltpu.SemaphoreType.DMA,
        pltpu.SemaphoreType.DMA,
        pltpu.SemaphoreType.DMA,
        pltpu.SemaphoreType.DMA,
        pltpu.SemaphoreType.DMA,
        pltpu.SemaphoreType.DMA,
        pltpu.SemaphoreType.DMA,
        pltpu.SemaphoreType.DMA,
        pltpu.SemaphoreType.DMA,
        pltpu.SemaphoreType.DMA,
    ],
    compiler_params=_SC_PARAMS,
)
def _edge_pass(hls_hbm, src_hbm, dst_hbm, attr_hbm, acc_hbm,
               srcw, dstw, attrw, dsta, buf0, buf1, buf2, buf3, acc_sh,
               semw0, semw1, semw2, semw3, semr0, semr1, semr2, semr3,
               sems0, sems1, sems2, sems3):
    c = lax.axis_index("c")
    s = lax.axis_index("s")
    wid = s * 2 + c
    bufs = (buf0, buf1, buf2, buf3)
    semw = (semw0, semw1, semw2, semw3)
    semr = (semr0, semr1, semr2, semr3)
    sems = (sems0, sems1, sems2, sems3)

    zz = jnp.zeros((L,), jnp.float32)

    def zbody(r, _):
        for cc in range(D // L):
            buf0[r, pl.ds(cc * L, L)] = zz
        return 0
    lax.fori_loop(0, KEB, zbody, 0)
    for half in range(RPS // KEB):
        pltpu.sync_copy(buf0, acc_sh.at[pl.ds(s * RPS + half * KEB, KEB)])
    plsc.subcore_barrier()

    def stage(g, p):
        eo = pl.ds(wid * EPW + g * KEB, KEB)
        pltpu.async_copy(src_hbm.at[eo], srcw.at[p], semw[p])
        pltpu.async_copy(dst_hbm.at[eo], dstw.at[p], semw[p])
        pltpu.async_copy(attr_hbm.at[eo], attrw.at[p], semw[p])

    def stage_wait(g, p):
        eo = pl.ds(wid * EPW + g * KEB, KEB)
        pltpu.make_async_copy(src_hbm.at[eo], srcw.at[p], semw[p]).wait()
        pltpu.make_async_copy(dst_hbm.at[eo], dstw.at[p], semw[p]).wait()
        pltpu.make_async_copy(attr_hbm.at[eo], attrw.at[p], semw[p]).wait()

    def fire(p):
        pltpu.async_copy(hls_hbm.at[srcw.at[p]], bufs[p], semr[p])

    def scat_wait(p):
        pltpu.make_async_copy(bufs[p], acc_sh.at[dsta.at[p]], sems[p]).wait()

    def proc(p):
        pltpu.make_async_copy(hls_hbm.at[srcw.at[p]], bufs[p], semr[p]).wait()

        def jb(k, _):
            av = attrw[p, pl.ds(k * L, L)]
            for jj in range(L):
                n = av[jj]
                row = k * L + jj
                for cc in range(D // L):
                    sl = pl.ds(cc * L, L)
                    bufs[p][row, sl] = n * bufs[p][row, sl]
            return 0
        lax.fori_loop(0, KEB // L, jb, 0)
        for k5 in range(KEB // L):
            dsta[p, pl.ds(k5 * L, L)] = dstw[p, pl.ds(k5 * L, L)]
        pltpu.async_copy(bufs[p], acc_sh.at[dsta.at[p]], sems[p], add=True)

    def step(g, ph):
        stage_wait(g + 1, (ph + 1) % NB)

        @pl.when(g >= NB - 1)
        def _():
            scat_wait((ph + 1) % NB)
        fire((ph + 1) % NB)

        @pl.when(g + 2 < NGB)
        def _():
            stage(g + 2, (ph + 2) % NB)
        proc(ph)

    stage(0, 0)
    stage(1, 1)
    stage_wait(0, 0)
    fire(0)

    def body(gq, _):
        for b in range(NB):
            step(gq * NB + b, b)
        return 0
    lax.fori_loop(0, (NGB - 1) // NB, body, 0)
    proc((NGB - 1) % NB)
    for ph in range(NB):
        scat_wait(ph)

    plsc.subcore_barrier()
    pltpu.sync_copy(acc_sh.at[pl.ds(s * RPS, RPS)],
                    acc_hbm.at[c, pl.ds(s * RPS, RPS)])


def _bn_linear(h, gamma, beta, W, dis):
    rowmask = lax.broadcasted_iota(jnp.int32, (NPAD, 1), 0) < N
    hm = jnp.where(rowmask, h, 0.0)
    mean = jnp.sum(hm, axis=0) / N
    dev = jnp.where(rowmask, h - mean, 0.0)
    var = jnp.sum(dev * dev, axis=0) / N
    hn = (h - mean) * lax.rsqrt(var + 1e-5) * gamma + beta
    hl = lax.dot_general(hn, W, (((1,), (1,)), ((), ())))
    return dis[:, None] * hl


def _tc1_body(h0_ref, degp_ref, g0_ref, be0_ref, W0_ref, dis_ref, hls_ref):
    deg = jnp.sum(degp_ref[...], axis=0)
    dis = jnp.where(deg > 0, lax.rsqrt(jnp.maximum(deg, 1e-12)), 0.0)
    dis_ref[...] = dis
    hls_ref[...] = _bn_linear(h0_ref[...], g0_ref[...], be0_ref[...],
                              W0_ref[...], dis)


_tc1 = pl.pallas_call(
    _tc1_body,
    out_shape=(
        jax.ShapeDtypeStruct((NPAD,), jnp.float32),
        jax.ShapeDtypeStruct((NPAD, D), jnp.float32),
    ),
)


def _tc2_body(acc_ref, dis_ref, b0_ref, g1_ref, be1_ref, W1_ref, hls_ref):
    dis = dis_ref[...]
    h = jnp.maximum(dis[:, None] * (acc_ref[0] + acc_ref[1]) + b0_ref[...],
                    0.0)
    hls_ref[...] = _bn_linear(h, g1_ref[...], be1_ref[...], W1_ref[...], dis)


_tc2 = pl.pallas_call(
    _tc2_body,
    out_shape=jax.ShapeDtypeStruct((NPAD, D), jnp.float32),
)


def _tc3_body(acc_ref, dis_ref, b1_ref, tf_ref, batch_ref, out_ref):
    dis = dis_ref[...]
    h = jnp.maximum(dis[:, None] * (acc_ref[0] + acc_ref[1]) + b1_ref[...],
                    0.0)
    batch = batch_ref[...]
    tf2 = tf_ref[...][:, None]
    gids = lax.broadcasted_iota(jnp.int32, (NPAD, G), 1)
    m = (batch[:, None] == gids).astype(jnp.float32)
    ex = jnp.exp(tf2)
    num = lax.dot_general(m, ex * h, (((0,), (0,)), ((), ())))
    den = lax.dot_general(m, ex, (((0,), (0,)), ((), ())))
    out_ref[...] = num / (den + 1e-16)


_tc3 = pl.pallas_call(
    _tc3_body,
    out_shape=jax.ShapeDtypeStruct((G, D), jnp.float32),
)


def kernel(x, edge_index, batch, edge_attr, emb_table,
           bn_gamma0, bn_beta0, W0, b0, bn_gamma1, bn_beta1, W1, b1):
    x_idx = x[:, 0].astype(jnp.int32)
    tfidf = x[:, 1]
    xp = jnp.pad(x_idx, (0, NPAD - N))
    src = edge_index[0].astype(jnp.int32)
    dst = edge_index[1].astype(jnp.int32)

    h0, degp = _emb_deg(xp, dst, edge_attr, emb_table)
    dis, hls0 = _tc1(h0, degp.reshape(NW, NPAD), bn_gamma0, bn_beta0, W0)
    acc0 = _edge_pass(hls0, src, dst, edge_attr)
    hls1 = _tc2(acc0, dis, b0, bn_gamma1, bn_beta1, W1)
    acc1 = _edge_pass(hls1, src, dst, edge_attr)

    batch_p = jnp.pad(batch.astype(jnp.int32), (0, NPAD - N),
                      constant_values=G)
    tf_p = jnp.pad(tfidf, (0, NPAD - N))
    return _tc3(acc1, dis, b1, tf_p, batch_p)
